# baseline (speedup 1.0000x reference)
import functools

import jax
import jax.numpy as jnp
from jax import lax
from jax.experimental import pallas as pl
from jax.experimental.pallas import tpu as pltpu
from jax.experimental.pallas import tpu_sc as plsc

VOCAB = 1000000
EMBED_DIM = 64
PAIR_ROWS = VOCAB // 2
PAIR_DIM = 128
BATCH = 16384
HIST_LEN = 50

NUM_CORES = 2
NUM_SUBCORES = 16
NUM_WORKERS = NUM_CORES * NUM_SUBCORES

B_PER_WORKER = BATCH // NUM_WORKERS
SUB = 2
B_CHUNK = B_PER_WORKER // SUB
VREGS_PER_CHUNK = B_CHUNK // 16

_mesh = plsc.VectorSubcoreMesh(
    core_axis_name="c", subcore_axis_name="s", num_cores=NUM_CORES
)


@functools.partial(
    pl.kernel,
    out_type=jax.ShapeDtypeStruct((HIST_LEN, EMBED_DIM, BATCH), jnp.float32),
    mesh=_mesh,
    scratch_types=[
        pltpu.VMEM((B_CHUNK,), jnp.int32),
        pltpu.VMEM((B_CHUNK,), jnp.int32),
        pltpu.VMEM((B_CHUNK,), jnp.int32),
        pltpu.VMEM((B_CHUNK,), jnp.int32),
        pltpu.VMEM((B_CHUNK,), jnp.int32),
        pltpu.VMEM((B_CHUNK,), jnp.int32),
        pltpu.VMEM((B_CHUNK, PAIR_DIM), jnp.float32),
        pltpu.VMEM((B_CHUNK, PAIR_DIM), jnp.float32),
        pltpu.VMEM((EMBED_DIM, B_CHUNK), jnp.float32),
        pltpu.SemaphoreType.DMA,
        pltpu.SemaphoreType.DMA,
    ],
    compiler_params=pltpu.CompilerParams(needs_layout_passes=False),
)
def _gather_kernel(table_hbm, idx_hbm, out_hbm, idx_0, idx_1, pair_0,
                   pair_1, half_0, half_1, g_0, g_1, t_v, sem0, sem1):
    wid = lax.axis_index("s") * NUM_CORES + lax.axis_index("c")
    wb = wid * B_PER_WORKER
    idx_bufs = (idx_0, idx_1)
    pair_bufs = (pair_0, pair_1)
    half_bufs = (half_0, half_1)
    g_bufs = (g_0, g_1)
    sems = (sem0, sem1)

    def stage(h, s):
        b0 = wb + s * B_CHUNK
        idx_v, pair_v, half_v = idx_bufs[s], pair_bufs[s], half_bufs[s]
        pltpu.sync_copy(idx_hbm.at[h, pl.ds(b0, B_CHUNK)], idx_v)

        def prep(v, c):
            sl = pl.ds(v * 16, 16)
            raw = idx_v[sl]
            pair_v[sl] = lax.shift_right_logical(raw, 1)
            half_v[sl] = lax.shift_left(lax.bitwise_and(raw, 1), 6)
            return c

        lax.fori_loop(0, VREGS_PER_CHUNK, prep, 0)
        pltpu.async_copy(table_hbm.at[pair_v], g_bufs[s], sems[s])

    def work(h, s):
        b0 = wb + s * B_CHUNK
        half_v = half_bufs[s]
        pltpu.make_async_copy(
            table_hbm.at[pair_bufs[s]], g_bufs[s], sems[s]
        ).wait()
        g = g_bufs[s]
        iota16 = lax.iota(jnp.int32, 16)

        def tr(v, c):
            rvec = v * 16 + iota16
            hvec = half_v[pl.ds(v * 16, 16)]
            for e0 in range(0, EMBED_DIM, 4):
                c0 = hvec + e0
                c1 = hvec + (e0 + 1)
                c2 = hvec + (e0 + 2)
                c3 = hvec + (e0 + 3)
                r0 = plsc.load_gather(g, [rvec, c0])
                r1 = plsc.load_gather(g, [rvec, c1])
                r2 = plsc.load_gather(g, [rvec, c2])
                r3 = plsc.load_gather(g, [rvec, c3])
                t_v[e0, pl.ds(v * 16, 16)] = r0
                t_v[e0 + 1, pl.ds(v * 16, 16)] = r1
                t_v[e0 + 2, pl.ds(v * 16, 16)] = r2
                t_v[e0 + 3, pl.ds(v * 16, 16)] = r3
            return c

        lax.fori_loop(0, VREGS_PER_CHUNK, tr, 0)
        pltpu.sync_copy(t_v, out_hbm.at[h, :, pl.ds(b0, B_CHUNK)])

    stage(0, 0)

    def h_body(h, carry):
        stage(h, 1)
        work(h, 0)

        @pl.when(h < HIST_LEN - 1)
        def _():
            stage(h + 1, 0)

        work(h, 1)
        return carry

    lax.fori_loop(0, HIST_LEN, h_body, 0)


def kernel(tf_matrix, items):
    table_pairs = tf_matrix.reshape(PAIR_ROWS, PAIR_DIM)
    items_t = items.T
    out = _gather_kernel(table_pairs, items_t)
    return out.transpose(2, 0, 1)


# speedup vs baseline: 1.2152x; 1.2152x over previous
import functools

import jax
import jax.numpy as jnp
from jax import lax
from jax.experimental import pallas as pl
from jax.experimental.pallas import tpu as pltpu
from jax.experimental.pallas import tpu_sc as plsc

VOCAB = 1000000
EMBED_DIM = 64
BATCH = 16384
HIST_LEN = 50

NUM_CORES = 2
NUM_SUBCORES = 16
NUM_WORKERS = NUM_CORES * NUM_SUBCORES

TOTAL_ROWS = BATCH * HIST_LEN
ROWS_PER_WORKER = TOTAL_ROWS // NUM_WORKERS
CHUNK = 800
NUM_CHUNKS = ROWS_PER_WORKER // CHUNK
OUTER = NUM_CHUNKS // 2

_mesh = plsc.VectorSubcoreMesh(
    core_axis_name="c", subcore_axis_name="s", num_cores=NUM_CORES
)


@functools.partial(
    pl.kernel,
    out_type=jax.ShapeDtypeStruct((TOTAL_ROWS, EMBED_DIM), jnp.float32),
    mesh=_mesh,
    scratch_types=[
        pltpu.VMEM((CHUNK,), jnp.int32),
        pltpu.VMEM((CHUNK,), jnp.int32),
        pltpu.VMEM((CHUNK, EMBED_DIM), jnp.float32),
        pltpu.VMEM((CHUNK, EMBED_DIM), jnp.float32),
        pltpu.SemaphoreType.DMA,
        pltpu.SemaphoreType.DMA,
        pltpu.SemaphoreType.DMA,
        pltpu.SemaphoreType.DMA,
    ],
    compiler_params=pltpu.CompilerParams(use_tc_tiling_on_sc=False),
)
def _gather_kernel(table_hbm, idx_hbm, out_hbm, idx_0, idx_1, g_0, g_1,
                   gsem0, gsem1, wsem0, wsem1):
    wid = lax.axis_index("s") * NUM_CORES + lax.axis_index("c")
    wbase = wid * ROWS_PER_WORKER
    idx_bufs = (idx_0, idx_1)
    g_bufs = (g_0, g_1)
    gsems = (gsem0, gsem1)
    wsems = (wsem0, wsem1)

    def stage(k, b):
        base = wbase + (2 * k + b) * CHUNK

        @pl.when(k > 0)
        def _():
            pltpu.make_async_copy(
                g_bufs[b], out_hbm.at[pl.ds(base, CHUNK)], wsems[b]
            ).wait()

        pltpu.sync_copy(idx_hbm.at[pl.ds(base, CHUNK)], idx_bufs[b])
        pltpu.async_copy(table_hbm.at[idx_bufs[b]], g_bufs[b], gsems[b])

    def work(k, b):
        base = wbase + (2 * k + b) * CHUNK
        pltpu.make_async_copy(
            table_hbm.at[idx_bufs[b]], g_bufs[b], gsems[b]
        ).wait()
        pltpu.async_copy(g_bufs[b], out_hbm.at[pl.ds(base, CHUNK)], wsems[b])

    stage(0, 0)

    def body(k, carry):
        stage(k, 1)
        work(k, 0)

        @pl.when(k < OUTER - 1)
        def _():
            stage(k + 1, 0)

        work(k, 1)
        return carry

    lax.fori_loop(0, OUTER, body, 0)

    last0 = wbase + (2 * (OUTER - 1)) * CHUNK
    last1 = wbase + (2 * (OUTER - 1) + 1) * CHUNK
    pltpu.make_async_copy(
        g_bufs[0], out_hbm.at[pl.ds(last0, CHUNK)], wsems[0]
    ).wait()
    pltpu.make_async_copy(
        g_bufs[1], out_hbm.at[pl.ds(last1, CHUNK)], wsems[1]
    ).wait()


def kernel(tf_matrix, items):
    flat_idx = items.reshape(-1)
    out = _gather_kernel(tf_matrix, flat_idx)
    return out.reshape(BATCH, HIST_LEN, EMBED_DIM)
